# Initial kernel scaffold; baseline (speedup 1.0000x reference)
import jax
import jax.numpy as jnp
from jax.experimental import pallas as pl


def kernel(x, edge_index, batch, W1, b1, W2, b2, W3, b3, Wfc1, bfc1, Wfc2, bfc2):
    raise NotImplementedError("write your pallas kernel here")


# speedup vs baseline: 10.2045x; 10.2045x over previous
import functools

import jax
import jax.numpy as jnp
from jax import lax
from jax.experimental import pallas as pl
from jax.experimental.pallas import tpu as pltpu
from jax.experimental.pallas import tpu_sc as plsc

f32 = jnp.float32
i32 = jnp.int32

N = 9990
E = 140000
G = 90
F_IN = 557

N_PAD = 10240
F_PAD = 560
G_PAD = 96
NC = 2
NS = 16
EB = 128
ENB = 35
E_CORE = NS * ENB * EB
AB = 64
ANB = 5
ROWS_PER_TILE = N_PAD // NS
G_PER_TILE = G_PAD // NS
BLK = 256
NBLK = N_PAD // BLK


def _sc_mesh():
    return plsc.VectorSubcoreMesh(
        core_axis_name="c", subcore_axis_name="s",
        num_cores=NC, num_subcores=NS)


def _sc_params():
    return pltpu.CompilerParams(use_tc_tiling_on_sc=False)


@functools.cache
def _deg_kernel_fn():
    @functools.partial(
        pl.kernel,
        out_type=jax.ShapeDtypeStruct((NC, N_PAD, 16), f32),
        mesh=_sc_mesh(),
        compiler_params=_sc_params(),
        scratch_types=[
            pltpu.VMEM_SHARED((N_PAD, 16), f32),
            pltpu.VMEM((ENB, EB), i32),
            pltpu.VMEM((EB, 16), f32),
        ],
    )
    def _deg_kernel(dst_hbm, ones_hbm, zeros_hbm, out_hbm, acc, dst_v, ones_v):
        c = lax.axis_index("c")
        s = lax.axis_index("s")
        r0 = s * ROWS_PER_TILE
        pltpu.sync_copy(zeros_hbm.at[pl.ds(r0, ROWS_PER_TILE)],
                        acc.at[pl.ds(r0, ROWS_PER_TILE)])
        pltpu.sync_copy(dst_hbm.at[c, s], dst_v)
        pltpu.sync_copy(ones_hbm, ones_v)
        plsc.subcore_barrier()

        @pl.loop(0, ENB)
        def _(j):
            pltpu.sync_copy(ones_v, acc.at[dst_v.at[j]], add=True)

        plsc.subcore_barrier()
        pltpu.sync_copy(acc.at[pl.ds(r0, ROWS_PER_TILE)],
                        out_hbm.at[c, pl.ds(r0, ROWS_PER_TILE)])

    return _deg_kernel


@functools.cache
def _make_edge_scatter(D):
    @functools.partial(
        pl.kernel,
        out_type=jax.ShapeDtypeStruct((NC, N_PAD, D), f32),
        mesh=_sc_mesh(),
        compiler_params=_sc_params(),
        scratch_types=[
            pltpu.VMEM_SHARED((N_PAD, D), f32),
            pltpu.VMEM((ENB, EB), i32),
            pltpu.VMEM((ENB, EB), i32),
            pltpu.VMEM((EB, D), f32),
        ],
    )
    def k(y_hbm, src_hbm, dst_hbm, zeros_hbm, out_hbm, acc, src_v, dst_v,
          rows_v):
        c = lax.axis_index("c")
        s = lax.axis_index("s")
        r0 = s * ROWS_PER_TILE
        pltpu.sync_copy(zeros_hbm.at[pl.ds(r0, ROWS_PER_TILE)],
                        acc.at[pl.ds(r0, ROWS_PER_TILE)])
        pltpu.sync_copy(src_hbm.at[c, s], src_v)
        pltpu.sync_copy(dst_hbm.at[c, s], dst_v)
        plsc.subcore_barrier()

        @pl.loop(0, ENB)
        def _(j):
            pltpu.sync_copy(y_hbm.at[src_v.at[j]], rows_v)
            pltpu.sync_copy(rows_v, acc.at[dst_v.at[j]], add=True)

        plsc.subcore_barrier()
        pltpu.sync_copy(acc.at[pl.ds(r0, ROWS_PER_TILE)],
                        out_hbm.at[c, pl.ds(r0, ROWS_PER_TILE)])

    return k


@functools.cache
def _agg_kernel_fn():
    @functools.partial(
        pl.kernel,
        out_type=[
            jax.ShapeDtypeStruct((NC, G_PAD, F_PAD), f32),
            jax.ShapeDtypeStruct((NC, G_PAD, 64), f32),
            jax.ShapeDtypeStruct((NC, G_PAD, 32), f32),
            jax.ShapeDtypeStruct((NC, G_PAD, 16), f32),
            jax.ShapeDtypeStruct((NC, G_PAD, 16), f32),
        ],
        mesh=_sc_mesh(),
        compiler_params=_sc_params(),
        scratch_types=[
        pltpu.VMEM_SHARED((G_PAD, F_PAD), f32),
        pltpu.VMEM_SHARED((G_PAD, 64), f32),
        pltpu.VMEM_SHARED((G_PAD, 32), f32),
        pltpu.VMEM_SHARED((G_PAD, 16), f32),
        pltpu.VMEM_SHARED((G_PAD, 16), f32),
        pltpu.VMEM((ANB, AB), i32),
        pltpu.VMEM((AB, F_PAD), f32),
        pltpu.VMEM((AB, 64), f32),
        pltpu.VMEM((AB, 32), f32),
        pltpu.VMEM((AB, 16), f32),
        pltpu.VMEM((AB, 16), f32),
    ],
)
    def _agg_kernel(x_hbm, h1_hbm, h2_hbm, h3_hbm, batch_hbm, ones_hbm,
                    zx_hbm, z64_hbm, z32_hbm, z16_hbm,
                    ox, o1, o2, o3, ocnt,
                    sx, s1, s2, s3, scnt, batch_v, bx, b1_, b2_, b3_, ones_v):
        c = lax.axis_index("c")
        s = lax.axis_index("s")
        wid = c * NS + s
        g0 = s * G_PER_TILE
        pltpu.sync_copy(zx_hbm.at[pl.ds(g0, G_PER_TILE)], sx.at[pl.ds(g0, G_PER_TILE)])
        pltpu.sync_copy(z64_hbm.at[pl.ds(g0, G_PER_TILE)], s1.at[pl.ds(g0, G_PER_TILE)])
        pltpu.sync_copy(z32_hbm.at[pl.ds(g0, G_PER_TILE)], s2.at[pl.ds(g0, G_PER_TILE)])
        pltpu.sync_copy(z16_hbm.at[pl.ds(g0, G_PER_TILE)], s3.at[pl.ds(g0, G_PER_TILE)])
        pltpu.sync_copy(z16_hbm.at[pl.ds(g0, G_PER_TILE)], scnt.at[pl.ds(g0, G_PER_TILE)])
        pltpu.sync_copy(batch_hbm.at[c, s], batch_v)
        pltpu.sync_copy(ones_hbm, ones_v)
        plsc.subcore_barrier()

        @pl.loop(0, ANB)
        def _(j):
            r = wid * (ANB * AB) + j * AB
            pltpu.sync_copy(x_hbm.at[pl.ds(r, AB)], bx)
            pltpu.sync_copy(bx, sx.at[batch_v.at[j]], add=True)
            pltpu.sync_copy(h1_hbm.at[pl.ds(r, AB)], b1_)
            pltpu.sync_copy(b1_, s1.at[batch_v.at[j]], add=True)
            pltpu.sync_copy(h2_hbm.at[pl.ds(r, AB)], b2_)
            pltpu.sync_copy(b2_, s2.at[batch_v.at[j]], add=True)
            pltpu.sync_copy(h3_hbm.at[pl.ds(r, AB)], b3_)
            pltpu.sync_copy(b3_, s3.at[batch_v.at[j]], add=True)
            pltpu.sync_copy(ones_v, scnt.at[batch_v.at[j]], add=True)

        plsc.subcore_barrier()
        gsl = pl.ds(g0, G_PER_TILE)
        pltpu.sync_copy(sx.at[gsl], ox.at[c, gsl])
        pltpu.sync_copy(s1.at[gsl], o1.at[c, gsl])
        pltpu.sync_copy(s2.at[gsl], o2.at[c, gsl])
        pltpu.sync_copy(s3.at[gsl], o3.at[c, gsl])
        pltpu.sync_copy(scnt.at[gsl], ocnt.at[c, gsl])

    return _agg_kernel


def _mm_body(x_ref, w_ref, o_ref):
    o_ref[...] = jnp.dot(x_ref[...], w_ref[...], preferred_element_type=f32)


def _mm1(x_pad, w):
    return pl.pallas_call(
        _mm_body,
        grid=(NBLK,),
        in_specs=[pl.BlockSpec((BLK, F_PAD), lambda i: (i, 0)),
                  pl.BlockSpec((F_PAD, 64), lambda i: (0, 0))],
        out_specs=pl.BlockSpec((BLK, 64), lambda i: (i, 0)),
        out_shape=jax.ShapeDtypeStruct((N_PAD, 64), f32),
    )(x_pad, w)


def _scale_body(xw_ref, degp_ref, o_ref):
    deg = degp_ref[0, :, 0:1] + degp_ref[1, :, 0:1] + 1.0
    o_ref[...] = lax.rsqrt(deg) * xw_ref[...]


def _scale(xw, degp):
    d = xw.shape[1]
    return pl.pallas_call(
        _scale_body,
        grid=(NBLK,),
        in_specs=[pl.BlockSpec((BLK, d), lambda i: (i, 0)),
                  pl.BlockSpec((2, BLK, 16), lambda i: (0, i, 0))],
        out_specs=pl.BlockSpec((BLK, d), lambda i: (i, 0)),
        out_shape=jax.ShapeDtypeStruct((N_PAD, d), f32),
    )(xw, degp)


def _consume_mm_body(acc_ref, y_ref, degp_ref, b_ref, w_ref, h_ref, yn_ref):
    deg = degp_ref[0, :, 0:1] + degp_ref[1, :, 0:1] + 1.0
    dinv = lax.rsqrt(deg)
    h = jnp.maximum(dinv * (acc_ref[0] + acc_ref[1] + y_ref[...]) + b_ref[...],
                    0.0)
    h_ref[...] = h
    yn_ref[...] = dinv * jnp.dot(h, w_ref[...], preferred_element_type=f32)


def _consume_mm(acc, y, degp, b, wt):
    d = y.shape[1]
    dn = wt.shape[1]
    return pl.pallas_call(
        _consume_mm_body,
        grid=(NBLK,),
        in_specs=[pl.BlockSpec((2, BLK, d), lambda i: (0, i, 0)),
                  pl.BlockSpec((BLK, d), lambda i: (i, 0)),
                  pl.BlockSpec((2, BLK, 16), lambda i: (0, i, 0)),
                  pl.BlockSpec((1, d), lambda i: (0, 0)),
                  pl.BlockSpec((d, dn), lambda i: (0, 0))],
        out_specs=[pl.BlockSpec((BLK, d), lambda i: (i, 0)),
                   pl.BlockSpec((BLK, dn), lambda i: (i, 0))],
        out_shape=[jax.ShapeDtypeStruct((N_PAD, d), f32),
                   jax.ShapeDtypeStruct((N_PAD, dn), f32)],
    )(acc, y, degp, b, wt)


def _consume_body(acc_ref, y_ref, degp_ref, b_ref, h_ref):
    deg = degp_ref[0, :, 0:1] + degp_ref[1, :, 0:1] + 1.0
    dinv = lax.rsqrt(deg)
    h_ref[...] = jnp.maximum(
        dinv * (acc_ref[0] + acc_ref[1] + y_ref[...]) + b_ref[...], 0.0)


def _consume(acc, y, degp, b):
    d = y.shape[1]
    return pl.pallas_call(
        _consume_body,
        grid=(NBLK,),
        in_specs=[pl.BlockSpec((2, BLK, d), lambda i: (0, i, 0)),
                  pl.BlockSpec((BLK, d), lambda i: (i, 0)),
                  pl.BlockSpec((2, BLK, 16), lambda i: (0, i, 0)),
                  pl.BlockSpec((1, d), lambda i: (0, 0))],
        out_specs=pl.BlockSpec((BLK, d), lambda i: (i, 0)),
        out_shape=jax.ShapeDtypeStruct((N_PAD, d), f32),
    )(acc, y, degp, b)


def _head_body(ax, a1, a2, a3, acnt, wx, wa, wb, wc, bf1, w2, bf2, o_ref):
    cnt = acnt[0, :, 0:1] + acnt[1, :, 0:1]
    scale = (1.0 / jnp.sqrt(1.0 + 1e-5)) / jnp.maximum(cnt, 1.0)
    mx = (ax[0] + ax[1]) * scale
    m1 = (a1[0] + a1[1]) * scale
    m2 = (a2[0] + a2[1]) * scale
    m3 = (a3[0] + a3[1]) * scale
    z = (jnp.dot(mx, wx[...], preferred_element_type=f32)
         + jnp.dot(m1, wa[...], preferred_element_type=f32)
         + jnp.dot(m2, wb[...], preferred_element_type=f32)
         + jnp.dot(m3, wc[...], preferred_element_type=f32)
         + bf1[...])
    z = jnp.maximum(z, 0.0)
    o_ref[...] = jax.nn.sigmoid(
        jnp.dot(z, w2[...], preferred_element_type=f32) + bf2[...])


def _head(ax, a1, a2, a3, acnt, wx, wa, wb, wc, bf1, w2, bf2):
    full = lambda shape: pl.BlockSpec(shape, lambda: tuple(0 for _ in shape))
    args = (ax, a1, a2, a3, acnt, wx, wa, wb, wc, bf1, w2, bf2)
    return pl.pallas_call(
        _head_body,
        in_specs=[full(a.shape) for a in args],
        out_specs=full((G_PAD, 1)),
        out_shape=jax.ShapeDtypeStruct((G_PAD, 1), f32),
    )(*args)


def kernel(x, edge_index, batch, W1, b1, W2, b2, W3, b3, Wfc1, bfc1, Wfc2,
           bfc2):
    x_pad = jnp.zeros((N_PAD, F_PAD), f32).at[:N, :F_IN].set(x)
    w1t = jnp.zeros((F_PAD, 64), f32).at[:F_IN].set(W1.T)

    half = E // NC

    def shard_edges(a):
        out = jnp.full((NC, E_CORE), N, i32)
        out = out.at[0, :half].set(a[:half]).at[1, :half].set(a[half:])
        return out.reshape(NC, NS, ENB, EB)

    srcs = shard_edges(edge_index[0])
    dsts = shard_edges(edge_index[1])
    batch_p = (jnp.full((N_PAD,), G, i32).at[:N].set(batch)
               .reshape(NC, NS, ANB, AB))

    ones_e = jnp.ones((EB, 16), f32)
    ones_a = jnp.ones((AB, 16), f32)
    z64 = jnp.zeros((N_PAD, 64), f32)
    z32 = jnp.zeros((N_PAD, 32), f32)
    z16 = jnp.zeros((N_PAD, 16), f32)
    zgx = jnp.zeros((G_PAD, F_PAD), f32)
    zg64 = jnp.zeros((G_PAD, 64), f32)
    zg32 = jnp.zeros((G_PAD, 32), f32)
    zg16 = jnp.zeros((G_PAD, 16), f32)

    degp = _deg_kernel_fn()(dsts, ones_e, z16)
    xw1 = _mm1(x_pad, w1t)
    y1 = _scale(xw1, degp)
    acc1 = _make_edge_scatter(64)(y1, srcs, dsts, z64)
    h1, y2 = _consume_mm(acc1, y1, degp, b1.reshape(1, 64), W2.T)
    acc2 = _make_edge_scatter(32)(y2, srcs, dsts, z32)
    h2, y3 = _consume_mm(acc2, y2, degp, b2.reshape(1, 32), W3.T)
    acc3 = _make_edge_scatter(16)(y3, srcs, dsts, z16)
    h3 = _consume(acc3, y3, degp, b3.reshape(1, 16))

    ax, a1, a2, a3, acnt = _agg_kernel_fn()(
        x_pad, h1, h2, h3, batch_p, ones_a, zgx, zg64, zg32, zg16)

    wx = jnp.zeros((F_PAD, 64), f32).at[:F_IN].set(Wfc1[:, :F_IN].T)
    wa = Wfc1[:, F_IN:F_IN + 64].T
    wb = Wfc1[:, F_IN + 64:F_IN + 96].T
    wc = Wfc1[:, F_IN + 96:].T
    out = _head(ax, a1, a2, a3, acnt, wx, wa, wb, wc,
                bfc1.reshape(1, 64), Wfc2.T, bfc2.reshape(1, 1))
    return out[:G]


# speedup vs baseline: 10.6006x; 1.0388x over previous
import functools

import jax
import jax.numpy as jnp
from jax import lax
from jax.experimental import pallas as pl
from jax.experimental.pallas import tpu as pltpu
from jax.experimental.pallas import tpu_sc as plsc

f32 = jnp.float32
i32 = jnp.int32

N = 9990
E = 140000
G = 90
F_IN = 557

N_PAD = 10240
F_PAD = 560
G_PAD = 96
NC = 2
NS = 16
EB = 128
ENB = 35
E_CORE = NS * ENB * EB
AB = 64
ANB = 5
ROWS_PER_TILE = N_PAD // NS
G_PER_TILE = G_PAD // NS
BLK = 256
NBLK = N_PAD // BLK


def _sc_mesh():
    return plsc.VectorSubcoreMesh(
        core_axis_name="c", subcore_axis_name="s",
        num_cores=NC, num_subcores=NS)


def _sc_params():
    return pltpu.CompilerParams(use_tc_tiling_on_sc=False)


@functools.cache
def _deg_kernel_fn():
    @functools.partial(
        pl.kernel,
        out_type=jax.ShapeDtypeStruct((NC, N_PAD, 16), f32),
        mesh=_sc_mesh(),
        compiler_params=_sc_params(),
        scratch_types=[
            pltpu.VMEM_SHARED((N_PAD, 16), f32),
            pltpu.VMEM((ENB, EB), i32),
            pltpu.VMEM((EB, 16), f32),
        ],
    )
    def _deg_kernel(dst_hbm, ones_hbm, zeros_hbm, out_hbm, acc, dst_v, ones_v):
        c = lax.axis_index("c")
        s = lax.axis_index("s")
        r0 = s * ROWS_PER_TILE
        pltpu.sync_copy(zeros_hbm.at[pl.ds(r0, ROWS_PER_TILE)],
                        acc.at[pl.ds(r0, ROWS_PER_TILE)])
        pltpu.sync_copy(dst_hbm.at[c, s], dst_v)
        pltpu.sync_copy(ones_hbm, ones_v)
        plsc.subcore_barrier()

        @pl.loop(0, ENB)
        def _(j):
            pltpu.sync_copy(ones_v, acc.at[dst_v.at[j]], add=True)

        plsc.subcore_barrier()
        pltpu.sync_copy(acc.at[pl.ds(r0, ROWS_PER_TILE)],
                        out_hbm.at[c, pl.ds(r0, ROWS_PER_TILE)])

    return _deg_kernel


@functools.cache
def _make_edge_scatter(D):
    @functools.partial(
        pl.kernel,
        out_type=jax.ShapeDtypeStruct((NC, N_PAD, D), f32),
        mesh=_sc_mesh(),
        compiler_params=_sc_params(),
        scratch_types=[
            pltpu.VMEM_SHARED((N_PAD, D), f32),
            pltpu.VMEM((ENB, EB), i32),
            pltpu.VMEM((ENB, EB), i32),
            pltpu.VMEM((EB, D), f32),
        ],
    )
    def k(y_hbm, src_hbm, dst_hbm, zeros_hbm, out_hbm, acc, src_v, dst_v,
          rows_v):
        c = lax.axis_index("c")
        s = lax.axis_index("s")
        r0 = s * ROWS_PER_TILE
        pltpu.sync_copy(zeros_hbm.at[pl.ds(r0, ROWS_PER_TILE)],
                        acc.at[pl.ds(r0, ROWS_PER_TILE)])
        pltpu.sync_copy(src_hbm.at[c, s], src_v)
        pltpu.sync_copy(dst_hbm.at[c, s], dst_v)
        plsc.subcore_barrier()

        @pl.loop(0, ENB)
        def _(j):
            pltpu.sync_copy(y_hbm.at[src_v.at[j]], rows_v)
            pltpu.sync_copy(rows_v, acc.at[dst_v.at[j]], add=True)

        plsc.subcore_barrier()
        pltpu.sync_copy(acc.at[pl.ds(r0, ROWS_PER_TILE)],
                        out_hbm.at[c, pl.ds(r0, ROWS_PER_TILE)])

    return k


@functools.cache
def _agg_kernel_fn():
    @functools.partial(
        pl.kernel,
        out_type=[
            jax.ShapeDtypeStruct((NC, G_PAD, 64), f32),
            jax.ShapeDtypeStruct((NC, G_PAD, 64), f32),
            jax.ShapeDtypeStruct((NC, G_PAD, 32), f32),
            jax.ShapeDtypeStruct((NC, G_PAD, 16), f32),
            jax.ShapeDtypeStruct((NC, G_PAD, 16), f32),
        ],
        mesh=_sc_mesh(),
        compiler_params=_sc_params(),
        scratch_types=[
        pltpu.VMEM_SHARED((G_PAD, 64), f32),
        pltpu.VMEM_SHARED((G_PAD, 64), f32),
        pltpu.VMEM_SHARED((G_PAD, 32), f32),
        pltpu.VMEM_SHARED((G_PAD, 16), f32),
        pltpu.VMEM_SHARED((G_PAD, 16), f32),
        pltpu.VMEM((ANB, AB), i32),
        pltpu.VMEM((AB, 64), f32),
        pltpu.VMEM((AB, 64), f32),
        pltpu.VMEM((AB, 32), f32),
        pltpu.VMEM((AB, 16), f32),
        pltpu.VMEM((AB, 16), f32),
    ],
)
    def _agg_kernel(x_hbm, h1_hbm, h2_hbm, h3_hbm, batch_hbm, ones_hbm,
                    zx_hbm, z64_hbm, z32_hbm, z16_hbm,
                    ox, o1, o2, o3, ocnt,
                    sx, s1, s2, s3, scnt, batch_v, bx, b1_, b2_, b3_, ones_v):
        c = lax.axis_index("c")
        s = lax.axis_index("s")
        wid = c * NS + s
        g0 = s * G_PER_TILE
        pltpu.sync_copy(zx_hbm.at[pl.ds(g0, G_PER_TILE)], sx.at[pl.ds(g0, G_PER_TILE)])
        pltpu.sync_copy(z64_hbm.at[pl.ds(g0, G_PER_TILE)], s1.at[pl.ds(g0, G_PER_TILE)])
        pltpu.sync_copy(z32_hbm.at[pl.ds(g0, G_PER_TILE)], s2.at[pl.ds(g0, G_PER_TILE)])
        pltpu.sync_copy(z16_hbm.at[pl.ds(g0, G_PER_TILE)], s3.at[pl.ds(g0, G_PER_TILE)])
        pltpu.sync_copy(z16_hbm.at[pl.ds(g0, G_PER_TILE)], scnt.at[pl.ds(g0, G_PER_TILE)])
        pltpu.sync_copy(batch_hbm.at[c, s], batch_v)
        pltpu.sync_copy(ones_hbm, ones_v)
        plsc.subcore_barrier()

        @pl.loop(0, ANB)
        def _(j):
            r = wid * (ANB * AB) + j * AB
            pltpu.sync_copy(x_hbm.at[pl.ds(r, AB)], bx)
            pltpu.sync_copy(bx, sx.at[batch_v.at[j]], add=True)
            pltpu.sync_copy(h1_hbm.at[pl.ds(r, AB)], b1_)
            pltpu.sync_copy(b1_, s1.at[batch_v.at[j]], add=True)
            pltpu.sync_copy(h2_hbm.at[pl.ds(r, AB)], b2_)
            pltpu.sync_copy(b2_, s2.at[batch_v.at[j]], add=True)
            pltpu.sync_copy(h3_hbm.at[pl.ds(r, AB)], b3_)
            pltpu.sync_copy(b3_, s3.at[batch_v.at[j]], add=True)
            pltpu.sync_copy(ones_v, scnt.at[batch_v.at[j]], add=True)

        plsc.subcore_barrier()
        gsl = pl.ds(g0, G_PER_TILE)
        pltpu.sync_copy(sx.at[gsl], ox.at[c, gsl])
        pltpu.sync_copy(s1.at[gsl], o1.at[c, gsl])
        pltpu.sync_copy(s2.at[gsl], o2.at[c, gsl])
        pltpu.sync_copy(s3.at[gsl], o3.at[c, gsl])
        pltpu.sync_copy(scnt.at[gsl], ocnt.at[c, gsl])

    return _agg_kernel


def _mm1_body(x_ref, w_ref, wx_ref, o_ref, oxp_ref):
    x = x_ref[...]
    o_ref[...] = jnp.dot(x, w_ref[...], preferred_element_type=f32)
    oxp_ref[...] = jnp.dot(x, wx_ref[...], preferred_element_type=f32)


def _mm1(x_pad, w, wx):
    return pl.pallas_call(
        _mm1_body,
        grid=(NBLK,),
        in_specs=[pl.BlockSpec((BLK, F_PAD), lambda i: (i, 0)),
                  pl.BlockSpec((F_PAD, 64), lambda i: (0, 0)),
                  pl.BlockSpec((F_PAD, 64), lambda i: (0, 0))],
        out_specs=[pl.BlockSpec((BLK, 64), lambda i: (i, 0)),
                   pl.BlockSpec((BLK, 64), lambda i: (i, 0))],
        out_shape=[jax.ShapeDtypeStruct((N_PAD, 64), f32),
                   jax.ShapeDtypeStruct((N_PAD, 64), f32)],
    )(x_pad, w, wx)


def _scale_body(xw_ref, degp_ref, o_ref):
    deg = degp_ref[0, :, 0:1] + degp_ref[1, :, 0:1] + 1.0
    o_ref[...] = lax.rsqrt(deg) * xw_ref[...]


def _scale(xw, degp):
    d = xw.shape[1]
    return pl.pallas_call(
        _scale_body,
        grid=(NBLK,),
        in_specs=[pl.BlockSpec((BLK, d), lambda i: (i, 0)),
                  pl.BlockSpec((2, BLK, 16), lambda i: (0, i, 0))],
        out_specs=pl.BlockSpec((BLK, d), lambda i: (i, 0)),
        out_shape=jax.ShapeDtypeStruct((N_PAD, d), f32),
    )(xw, degp)


def _consume_mm_body(acc_ref, y_ref, degp_ref, b_ref, w_ref, h_ref, yn_ref):
    deg = degp_ref[0, :, 0:1] + degp_ref[1, :, 0:1] + 1.0
    dinv = lax.rsqrt(deg)
    h = jnp.maximum(dinv * (acc_ref[0] + acc_ref[1] + y_ref[...]) + b_ref[...],
                    0.0)
    h_ref[...] = h
    yn_ref[...] = dinv * jnp.dot(h, w_ref[...], preferred_element_type=f32)


def _consume_mm(acc, y, degp, b, wt):
    d = y.shape[1]
    dn = wt.shape[1]
    return pl.pallas_call(
        _consume_mm_body,
        grid=(NBLK,),
        in_specs=[pl.BlockSpec((2, BLK, d), lambda i: (0, i, 0)),
                  pl.BlockSpec((BLK, d), lambda i: (i, 0)),
                  pl.BlockSpec((2, BLK, 16), lambda i: (0, i, 0)),
                  pl.BlockSpec((1, d), lambda i: (0, 0)),
                  pl.BlockSpec((d, dn), lambda i: (0, 0))],
        out_specs=[pl.BlockSpec((BLK, d), lambda i: (i, 0)),
                   pl.BlockSpec((BLK, dn), lambda i: (i, 0))],
        out_shape=[jax.ShapeDtypeStruct((N_PAD, d), f32),
                   jax.ShapeDtypeStruct((N_PAD, dn), f32)],
    )(acc, y, degp, b, wt)


def _consume_body(acc_ref, y_ref, degp_ref, b_ref, h_ref):
    deg = degp_ref[0, :, 0:1] + degp_ref[1, :, 0:1] + 1.0
    dinv = lax.rsqrt(deg)
    h_ref[...] = jnp.maximum(
        dinv * (acc_ref[0] + acc_ref[1] + y_ref[...]) + b_ref[...], 0.0)


def _consume(acc, y, degp, b):
    d = y.shape[1]
    return pl.pallas_call(
        _consume_body,
        grid=(NBLK,),
        in_specs=[pl.BlockSpec((2, BLK, d), lambda i: (0, i, 0)),
                  pl.BlockSpec((BLK, d), lambda i: (i, 0)),
                  pl.BlockSpec((2, BLK, 16), lambda i: (0, i, 0)),
                  pl.BlockSpec((1, d), lambda i: (0, 0))],
        out_specs=pl.BlockSpec((BLK, d), lambda i: (i, 0)),
        out_shape=jax.ShapeDtypeStruct((N_PAD, d), f32),
    )(acc, y, degp, b)


def _head_body(axp, a1, a2, a3, acnt, wa, wb, wc, bf1, w2, bf2, o_ref):
    cnt = acnt[0, :, 0:1] + acnt[1, :, 0:1]
    scale = (1.0 / jnp.sqrt(1.0 + 1e-5)) / jnp.maximum(cnt, 1.0)
    m1 = (a1[0] + a1[1]) * scale
    m2 = (a2[0] + a2[1]) * scale
    m3 = (a3[0] + a3[1]) * scale
    z = ((axp[0] + axp[1]) * scale
         + jnp.dot(m1, wa[...], preferred_element_type=f32)
         + jnp.dot(m2, wb[...], preferred_element_type=f32)
         + jnp.dot(m3, wc[...], preferred_element_type=f32)
         + bf1[...])
    z = jnp.maximum(z, 0.0)
    o_ref[...] = jax.nn.sigmoid(
        jnp.dot(z, w2[...], preferred_element_type=f32) + bf2[...])


def _head(axp, a1, a2, a3, acnt, wa, wb, wc, bf1, w2, bf2):
    full = lambda shape: pl.BlockSpec(shape, lambda: tuple(0 for _ in shape))
    args = (axp, a1, a2, a3, acnt, wa, wb, wc, bf1, w2, bf2)
    return pl.pallas_call(
        _head_body,
        in_specs=[full(a.shape) for a in args],
        out_specs=full((G_PAD, 1)),
        out_shape=jax.ShapeDtypeStruct((G_PAD, 1), f32),
    )(*args)


def kernel(x, edge_index, batch, W1, b1, W2, b2, W3, b3, Wfc1, bfc1, Wfc2,
           bfc2):
    x_pad = jnp.zeros((N_PAD, F_PAD), f32).at[:N, :F_IN].set(x)
    w1t = jnp.zeros((F_PAD, 64), f32).at[:F_IN].set(W1.T)

    half = E // NC

    def shard_edges(a):
        out = jnp.full((NC, E_CORE), N, i32)
        out = out.at[0, :half].set(a[:half]).at[1, :half].set(a[half:])
        return out.reshape(NC, NS, ENB, EB)

    srcs = shard_edges(edge_index[0])
    dsts = shard_edges(edge_index[1])
    batch_p = (jnp.full((N_PAD,), G, i32).at[:N].set(batch)
               .reshape(NC, NS, ANB, AB))

    ones_e = jnp.ones((EB, 16), f32)
    ones_a = jnp.ones((AB, 16), f32)
    z64 = jnp.zeros((N_PAD, 64), f32)
    z32 = jnp.zeros((N_PAD, 32), f32)
    z16 = jnp.zeros((N_PAD, 16), f32)
    zg64 = jnp.zeros((G_PAD, 64), f32)
    zg32 = jnp.zeros((G_PAD, 32), f32)
    zg16 = jnp.zeros((G_PAD, 16), f32)

    wx = jnp.zeros((F_PAD, 64), f32).at[:F_IN].set(Wfc1[:, :F_IN].T)

    degp = _deg_kernel_fn()(dsts, ones_e, z16)
    xw1, xp = _mm1(x_pad, w1t, wx)
    y1 = _scale(xw1, degp)
    acc1 = _make_edge_scatter(64)(y1, srcs, dsts, z64)
    h1, y2 = _consume_mm(acc1, y1, degp, b1.reshape(1, 64), W2.T)
    acc2 = _make_edge_scatter(32)(y2, srcs, dsts, z32)
    h2, y3 = _consume_mm(acc2, y2, degp, b2.reshape(1, 32), W3.T)
    acc3 = _make_edge_scatter(16)(y3, srcs, dsts, z16)
    h3 = _consume(acc3, y3, degp, b3.reshape(1, 16))

    axp, a1, a2, a3, acnt = _agg_kernel_fn()(
        xp, h1, h2, h3, batch_p, ones_a, zg64, zg64, zg32, zg16)

    wa = Wfc1[:, F_IN:F_IN + 64].T
    wb = Wfc1[:, F_IN + 64:F_IN + 96].T
    wc = Wfc1[:, F_IN + 96:].T
    out = _head(axp, a1, a2, a3, acnt, wa, wb, wc,
                bfc1.reshape(1, 64), Wfc2.T, bfc2.reshape(1, 1))
    return out[:G]


# speedup vs baseline: 12.3280x; 1.1630x over previous
import functools

import jax
import jax.numpy as jnp
from jax import lax
from jax.experimental import pallas as pl
from jax.experimental.pallas import tpu as pltpu
from jax.experimental.pallas import tpu_sc as plsc

f32 = jnp.float32
i32 = jnp.int32

N = 9990
E = 140000
G = 90
F_IN = 557

N_PAD = 10240
F_PAD = 560
G_PAD = 96
NC = 2
NS = 16
EB = 128
ENB = 35
E_CORE = NS * ENB * EB
AB = 64
ANB = 5
ROWS_PER_TILE = N_PAD // NS
G_PER_TILE = G_PAD // NS
BLK = 256
NBLK = N_PAD // BLK


def _sc_mesh():
    return plsc.VectorSubcoreMesh(
        core_axis_name="c", subcore_axis_name="s",
        num_cores=NC, num_subcores=NS)


def _sc_params():
    return pltpu.CompilerParams(use_tc_tiling_on_sc=False)


@functools.cache
def _deg_kernel_fn():
    @functools.partial(
        pl.kernel,
        out_type=jax.ShapeDtypeStruct((NC, N_PAD, 16), f32),
        mesh=_sc_mesh(),
        compiler_params=_sc_params(),
        scratch_types=[
            pltpu.VMEM_SHARED((N_PAD, 16), f32),
            pltpu.VMEM((ENB, EB), i32),
            pltpu.VMEM((EB, 16), f32),
        ],
    )
    def _deg_kernel(dst_hbm, ones_hbm, zeros_hbm, out_hbm, acc, dst_v, ones_v):
        c = lax.axis_index("c")
        s = lax.axis_index("s")
        r0 = s * ROWS_PER_TILE
        pltpu.sync_copy(zeros_hbm.at[pl.ds(r0, ROWS_PER_TILE)],
                        acc.at[pl.ds(r0, ROWS_PER_TILE)])
        pltpu.sync_copy(dst_hbm.at[c, s], dst_v)
        pltpu.sync_copy(ones_hbm, ones_v)
        plsc.subcore_barrier()

        @pl.loop(0, ENB)
        def _(j):
            pltpu.sync_copy(ones_v, acc.at[dst_v.at[j]], add=True)

        plsc.subcore_barrier()
        pltpu.sync_copy(acc.at[pl.ds(r0, ROWS_PER_TILE)],
                        out_hbm.at[c, pl.ds(r0, ROWS_PER_TILE)])

    return _deg_kernel


@functools.cache
def _make_edge_scatter(D):
    @functools.partial(
        pl.kernel,
        out_type=jax.ShapeDtypeStruct((NC, N_PAD, D), f32),
        mesh=_sc_mesh(),
        compiler_params=_sc_params(),
        scratch_types=[
            pltpu.VMEM_SHARED((N_PAD, D), f32),
            pltpu.VMEM((ENB, EB), i32),
            pltpu.VMEM((ENB, EB), i32),
            pltpu.VMEM((EB, D), f32),
        ],
    )
    def k(y_hbm, src_hbm, dst_hbm, zeros_hbm, out_hbm, acc, src_v, dst_v,
          rows_v):
        c = lax.axis_index("c")
        s = lax.axis_index("s")
        r0 = s * ROWS_PER_TILE
        pltpu.sync_copy(zeros_hbm.at[pl.ds(r0, ROWS_PER_TILE)],
                        acc.at[pl.ds(r0, ROWS_PER_TILE)])
        pltpu.sync_copy(src_hbm.at[c, s], src_v)
        pltpu.sync_copy(dst_hbm.at[c, s], dst_v)
        plsc.subcore_barrier()

        @pl.loop(0, ENB)
        def _(j):
            pltpu.sync_copy(y_hbm.at[src_v.at[j]], rows_v)
            pltpu.sync_copy(rows_v, acc.at[dst_v.at[j]], add=True)

        plsc.subcore_barrier()
        pltpu.sync_copy(acc.at[pl.ds(r0, ROWS_PER_TILE)],
                        out_hbm.at[c, pl.ds(r0, ROWS_PER_TILE)])

    return k


@functools.cache
def _agg_kernel_fn():
    @functools.partial(
        pl.kernel,
        out_type=[
            jax.ShapeDtypeStruct((NC, G_PAD, 64), f32),
            jax.ShapeDtypeStruct((NC, G_PAD, 64), f32),
            jax.ShapeDtypeStruct((NC, G_PAD, 32), f32),
            jax.ShapeDtypeStruct((NC, G_PAD, 16), f32),
            jax.ShapeDtypeStruct((NC, G_PAD, 16), f32),
        ],
        mesh=_sc_mesh(),
        compiler_params=_sc_params(),
        scratch_types=[
        pltpu.VMEM_SHARED((G_PAD, 64), f32),
        pltpu.VMEM_SHARED((G_PAD, 64), f32),
        pltpu.VMEM_SHARED((G_PAD, 32), f32),
        pltpu.VMEM_SHARED((G_PAD, 16), f32),
        pltpu.VMEM_SHARED((G_PAD, 16), f32),
        pltpu.VMEM((ANB, AB), i32),
        pltpu.VMEM((AB, 64), f32),
        pltpu.VMEM((AB, 64), f32),
        pltpu.VMEM((AB, 32), f32),
        pltpu.VMEM((AB, 16), f32),
        pltpu.VMEM((AB, 16), f32),
    ],
)
    def _agg_kernel(x_hbm, h1_hbm, h2_hbm, h3_hbm, batch_hbm, ones_hbm,
                    zx_hbm, z64_hbm, z32_hbm, z16_hbm,
                    ox, o1, o2, o3, ocnt,
                    sx, s1, s2, s3, scnt, batch_v, bx, b1_, b2_, b3_, ones_v):
        c = lax.axis_index("c")
        s = lax.axis_index("s")
        wid = c * NS + s
        g0 = s * G_PER_TILE
        pltpu.sync_copy(zx_hbm.at[pl.ds(g0, G_PER_TILE)], sx.at[pl.ds(g0, G_PER_TILE)])
        pltpu.sync_copy(z64_hbm.at[pl.ds(g0, G_PER_TILE)], s1.at[pl.ds(g0, G_PER_TILE)])
        pltpu.sync_copy(z32_hbm.at[pl.ds(g0, G_PER_TILE)], s2.at[pl.ds(g0, G_PER_TILE)])
        pltpu.sync_copy(z16_hbm.at[pl.ds(g0, G_PER_TILE)], s3.at[pl.ds(g0, G_PER_TILE)])
        pltpu.sync_copy(z16_hbm.at[pl.ds(g0, G_PER_TILE)], scnt.at[pl.ds(g0, G_PER_TILE)])
        pltpu.sync_copy(batch_hbm.at[c, s], batch_v)
        pltpu.sync_copy(ones_hbm, ones_v)
        plsc.subcore_barrier()

        @pl.loop(0, ANB)
        def _(j):
            r = wid * (ANB * AB) + j * AB
            pltpu.sync_copy(x_hbm.at[pl.ds(r, AB)], bx)
            pltpu.sync_copy(bx, sx.at[batch_v.at[j]], add=True)
            pltpu.sync_copy(h1_hbm.at[pl.ds(r, AB)], b1_)
            pltpu.sync_copy(b1_, s1.at[batch_v.at[j]], add=True)
            pltpu.sync_copy(h2_hbm.at[pl.ds(r, AB)], b2_)
            pltpu.sync_copy(b2_, s2.at[batch_v.at[j]], add=True)
            pltpu.sync_copy(h3_hbm.at[pl.ds(r, AB)], b3_)
            pltpu.sync_copy(b3_, s3.at[batch_v.at[j]], add=True)
            pltpu.sync_copy(ones_v, scnt.at[batch_v.at[j]], add=True)

        plsc.subcore_barrier()
        gsl = pl.ds(g0, G_PER_TILE)
        pltpu.sync_copy(sx.at[gsl], ox.at[c, gsl])
        pltpu.sync_copy(s1.at[gsl], o1.at[c, gsl])
        pltpu.sync_copy(s2.at[gsl], o2.at[c, gsl])
        pltpu.sync_copy(s3.at[gsl], o3.at[c, gsl])
        pltpu.sync_copy(scnt.at[gsl], ocnt.at[c, gsl])

    return _agg_kernel


def _mm1_body(x_ref, w_ref, wx_ref, o_ref, oxp_ref):
    x = x_ref[...]
    o_ref[...] = jnp.dot(x, w_ref[...], preferred_element_type=f32)
    oxp_ref[...] = jnp.dot(x, wx_ref[...], preferred_element_type=f32)


def _mm1(x, w, wx):
    return pl.pallas_call(
        _mm1_body,
        grid=(NBLK,),
        in_specs=[pl.BlockSpec((BLK, F_IN), lambda i: (i, 0)),
                  pl.BlockSpec((F_IN, 64), lambda i: (0, 0)),
                  pl.BlockSpec((F_IN, 64), lambda i: (0, 0))],
        out_specs=[pl.BlockSpec((BLK, 64), lambda i: (i, 0)),
                   pl.BlockSpec((BLK, 64), lambda i: (i, 0))],
        out_shape=[jax.ShapeDtypeStruct((N_PAD, 64), f32),
                   jax.ShapeDtypeStruct((N_PAD, 64), f32)],
    )(x, w, wx)


def _scale_body(xw_ref, degp_ref, o_ref):
    deg = degp_ref[0, :, 0:1] + degp_ref[1, :, 0:1] + 1.0
    o_ref[...] = lax.rsqrt(deg) * xw_ref[...]


def _scale(xw, degp):
    d = xw.shape[1]
    return pl.pallas_call(
        _scale_body,
        grid=(NBLK,),
        in_specs=[pl.BlockSpec((BLK, d), lambda i: (i, 0)),
                  pl.BlockSpec((2, BLK, 16), lambda i: (0, i, 0))],
        out_specs=pl.BlockSpec((BLK, d), lambda i: (i, 0)),
        out_shape=jax.ShapeDtypeStruct((N_PAD, d), f32),
    )(xw, degp)


def _consume_mm_body(acc_ref, y_ref, degp_ref, b_ref, w_ref, h_ref, yn_ref):
    deg = degp_ref[0, :, 0:1] + degp_ref[1, :, 0:1] + 1.0
    dinv = lax.rsqrt(deg)
    h = jnp.maximum(dinv * (acc_ref[0] + acc_ref[1] + y_ref[...]) + b_ref[...],
                    0.0)
    h_ref[...] = h
    yn_ref[...] = dinv * jnp.dot(h, w_ref[...], preferred_element_type=f32)


def _consume_mm(acc, y, degp, b, wt):
    d = y.shape[1]
    dn = wt.shape[1]
    return pl.pallas_call(
        _consume_mm_body,
        grid=(NBLK,),
        in_specs=[pl.BlockSpec((2, BLK, d), lambda i: (0, i, 0)),
                  pl.BlockSpec((BLK, d), lambda i: (i, 0)),
                  pl.BlockSpec((2, BLK, 16), lambda i: (0, i, 0)),
                  pl.BlockSpec((1, d), lambda i: (0, 0)),
                  pl.BlockSpec((d, dn), lambda i: (0, 0))],
        out_specs=[pl.BlockSpec((BLK, d), lambda i: (i, 0)),
                   pl.BlockSpec((BLK, dn), lambda i: (i, 0))],
        out_shape=[jax.ShapeDtypeStruct((N_PAD, d), f32),
                   jax.ShapeDtypeStruct((N_PAD, dn), f32)],
    )(acc, y, degp, b, wt)


def _consume_body(acc_ref, y_ref, degp_ref, b_ref, h_ref):
    deg = degp_ref[0, :, 0:1] + degp_ref[1, :, 0:1] + 1.0
    dinv = lax.rsqrt(deg)
    h_ref[...] = jnp.maximum(
        dinv * (acc_ref[0] + acc_ref[1] + y_ref[...]) + b_ref[...], 0.0)


def _consume(acc, y, degp, b):
    d = y.shape[1]
    return pl.pallas_call(
        _consume_body,
        grid=(NBLK,),
        in_specs=[pl.BlockSpec((2, BLK, d), lambda i: (0, i, 0)),
                  pl.BlockSpec((BLK, d), lambda i: (i, 0)),
                  pl.BlockSpec((2, BLK, 16), lambda i: (0, i, 0)),
                  pl.BlockSpec((1, d), lambda i: (0, 0))],
        out_specs=pl.BlockSpec((BLK, d), lambda i: (i, 0)),
        out_shape=jax.ShapeDtypeStruct((N_PAD, d), f32),
    )(acc, y, degp, b)


def _head_body(axp, a1, a2, a3, acnt, wa, wb, wc, bf1, w2, bf2, o_ref):
    cnt = acnt[0, :, 0:1] + acnt[1, :, 0:1]
    scale = (1.0 / jnp.sqrt(1.0 + 1e-5)) / jnp.maximum(cnt, 1.0)
    m1 = (a1[0] + a1[1]) * scale
    m2 = (a2[0] + a2[1]) * scale
    m3 = (a3[0] + a3[1]) * scale
    z = ((axp[0] + axp[1]) * scale
         + jnp.dot(m1, wa[...], preferred_element_type=f32)
         + jnp.dot(m2, wb[...], preferred_element_type=f32)
         + jnp.dot(m3, wc[...], preferred_element_type=f32)
         + bf1[...])
    z = jnp.maximum(z, 0.0)
    o_ref[...] = jax.nn.sigmoid(
        jnp.dot(z, w2[...], preferred_element_type=f32) + bf2[...])


def _head(axp, a1, a2, a3, acnt, wa, wb, wc, bf1, w2, bf2):
    full = lambda shape: pl.BlockSpec(shape, lambda: tuple(0 for _ in shape))
    args = (axp, a1, a2, a3, acnt, wa, wb, wc, bf1, w2, bf2)
    return pl.pallas_call(
        _head_body,
        in_specs=[full(a.shape) for a in args],
        out_specs=full((G_PAD, 1)),
        out_shape=jax.ShapeDtypeStruct((G_PAD, 1), f32),
    )(*args)


def kernel(x, edge_index, batch, W1, b1, W2, b2, W3, b3, Wfc1, bfc1, Wfc2,
           bfc2):
    w1t = W1.T

    half = E // NC

    def shard_edges(a):
        out = jnp.full((NC, E_CORE), N, i32)
        out = out.at[0, :half].set(a[:half]).at[1, :half].set(a[half:])
        return out.reshape(NC, NS, ENB, EB)

    srcs = shard_edges(edge_index[0])
    dsts = shard_edges(edge_index[1])
    batch_p = (jnp.full((N_PAD,), G, i32).at[:N].set(batch)
               .reshape(NC, NS, ANB, AB))

    ones_e = jnp.ones((EB, 16), f32)
    ones_a = jnp.ones((AB, 16), f32)
    z64 = jnp.zeros((N_PAD, 64), f32)
    z32 = jnp.zeros((N_PAD, 32), f32)
    z16 = jnp.zeros((N_PAD, 16), f32)
    zg64 = jnp.zeros((G_PAD, 64), f32)
    zg32 = jnp.zeros((G_PAD, 32), f32)
    zg16 = jnp.zeros((G_PAD, 16), f32)

    wx = Wfc1[:, :F_IN].T

    degp = _deg_kernel_fn()(dsts, ones_e, z16)
    xw1, xp = _mm1(x, w1t, wx)
    y1 = _scale(xw1, degp)
    acc1 = _make_edge_scatter(64)(y1, srcs, dsts, z64)
    h1, y2 = _consume_mm(acc1, y1, degp, b1.reshape(1, 64), W2.T)
    acc2 = _make_edge_scatter(32)(y2, srcs, dsts, z32)
    h2, y3 = _consume_mm(acc2, y2, degp, b2.reshape(1, 32), W3.T)
    acc3 = _make_edge_scatter(16)(y3, srcs, dsts, z16)
    h3 = _consume(acc3, y3, degp, b3.reshape(1, 16))

    axp, a1, a2, a3, acnt = _agg_kernel_fn()(
        xp, h1, h2, h3, batch_p, ones_a, zg64, zg64, zg32, zg16)

    wa = Wfc1[:, F_IN:F_IN + 64].T
    wb = Wfc1[:, F_IN + 64:F_IN + 96].T
    wc = Wfc1[:, F_IN + 96:].T
    out = _head(axp, a1, a2, a3, acnt, wa, wb, wc,
                bfc1.reshape(1, 64), Wfc2.T, bfc2.reshape(1, 1))
    return out[:G]


# speedup vs baseline: 18.6881x; 1.5159x over previous
import functools

import jax
import jax.numpy as jnp
from jax import lax
from jax.experimental import pallas as pl
from jax.experimental.pallas import tpu as pltpu
from jax.experimental.pallas import tpu_sc as plsc

f32 = jnp.float32
i32 = jnp.int32

N = 9990
E = 140000
G = 90
F_IN = 557

N_PAD = 10240
G_PAD = 96
NC = 2
NS = 16
EB = 125
ENB = 35
E_CORE = NS * ENB * EB
AB = 64
ANB = 5
ROWS_PER_TILE = N_PAD // NS
G_PER_TILE = G_PAD // NS
BLK = 1024
NBLK = N_PAD // BLK


def _sc_mesh():
    return plsc.VectorSubcoreMesh(
        core_axis_name="c", subcore_axis_name="s",
        num_cores=NC, num_subcores=NS)


def _sc_params():
    return pltpu.CompilerParams(use_tc_tiling_on_sc=False)


@functools.cache
def _deg_kernel_fn():
    @functools.partial(
        pl.kernel,
        out_type=jax.ShapeDtypeStruct((NC, N_PAD, 16), f32),
        mesh=_sc_mesh(),
        compiler_params=_sc_params(),
        scratch_types=[
            pltpu.VMEM_SHARED((N_PAD, 16), f32),
            pltpu.VMEM((ENB, EB), i32),
            pltpu.VMEM((EB, 16), f32),
        ],
    )
    def _deg_kernel(dst_hbm, ones_hbm, zeros_hbm, out_hbm, acc, dst_v, ones_v):
        c = lax.axis_index("c")
        s = lax.axis_index("s")
        r0 = s * ROWS_PER_TILE
        pltpu.sync_copy(zeros_hbm.at[pl.ds(r0, ROWS_PER_TILE)],
                        acc.at[pl.ds(r0, ROWS_PER_TILE)])
        pltpu.sync_copy(dst_hbm.at[c, s], dst_v)
        pltpu.sync_copy(ones_hbm, ones_v)
        plsc.subcore_barrier()

        @pl.loop(0, ENB)
        def _(j):
            pltpu.sync_copy(ones_v, acc.at[dst_v.at[j]], add=True)

        plsc.subcore_barrier()
        pltpu.sync_copy(acc.at[pl.ds(r0, ROWS_PER_TILE)],
                        out_hbm.at[c, pl.ds(r0, ROWS_PER_TILE)])

    return _deg_kernel


@functools.cache
def _make_edge_scatter(D):
    @functools.partial(
        pl.kernel,
        out_type=jax.ShapeDtypeStruct((NC, N_PAD, D), f32),
        mesh=_sc_mesh(),
        compiler_params=_sc_params(),
        scratch_types=[
            pltpu.VMEM_SHARED((N_PAD, D), f32),
            pltpu.VMEM((ENB, EB), i32),
            pltpu.VMEM((ENB, EB), i32),
            pltpu.VMEM((EB, D), f32),
        ],
    )
    def k(y_hbm, src_hbm, dst_hbm, zeros_hbm, out_hbm, acc, src_v, dst_v,
          rows_v):
        c = lax.axis_index("c")
        s = lax.axis_index("s")
        r0 = s * ROWS_PER_TILE
        pltpu.sync_copy(zeros_hbm.at[pl.ds(r0, ROWS_PER_TILE)],
                        acc.at[pl.ds(r0, ROWS_PER_TILE)])
        pltpu.sync_copy(src_hbm.at[c, s], src_v)
        pltpu.sync_copy(dst_hbm.at[c, s], dst_v)
        plsc.subcore_barrier()

        @pl.loop(0, ENB)
        def _(j):
            pltpu.sync_copy(y_hbm.at[src_v.at[j]], rows_v)
            pltpu.sync_copy(rows_v, acc.at[dst_v.at[j]], add=True)

        plsc.subcore_barrier()
        pltpu.sync_copy(acc.at[pl.ds(r0, ROWS_PER_TILE)],
                        out_hbm.at[c, pl.ds(r0, ROWS_PER_TILE)])

    return k


@functools.cache
def _agg_kernel_fn():
    @functools.partial(
        pl.kernel,
        out_type=[
            jax.ShapeDtypeStruct((NC, G_PAD, 64), f32),
            jax.ShapeDtypeStruct((NC, G_PAD, 64), f32),
            jax.ShapeDtypeStruct((NC, G_PAD, 32), f32),
            jax.ShapeDtypeStruct((NC, G_PAD, 16), f32),
            jax.ShapeDtypeStruct((NC, G_PAD, 16), f32),
        ],
        mesh=_sc_mesh(),
        compiler_params=_sc_params(),
        scratch_types=[
        pltpu.VMEM_SHARED((G_PAD, 64), f32),
        pltpu.VMEM_SHARED((G_PAD, 64), f32),
        pltpu.VMEM_SHARED((G_PAD, 32), f32),
        pltpu.VMEM_SHARED((G_PAD, 16), f32),
        pltpu.VMEM_SHARED((G_PAD, 16), f32),
        pltpu.VMEM((ANB, AB), i32),
        pltpu.VMEM((AB, 64), f32),
        pltpu.VMEM((AB, 64), f32),
        pltpu.VMEM((AB, 32), f32),
        pltpu.VMEM((AB, 16), f32),
        pltpu.VMEM((AB, 16), f32),
    ],
)
    def _agg_kernel(x_hbm, h1_hbm, h2_hbm, h3_hbm, batch_hbm, ones_hbm,
                    zx_hbm, z64_hbm, z32_hbm, z16_hbm,
                    ox, o1, o2, o3, ocnt,
                    sx, s1, s2, s3, scnt, batch_v, bx, b1_, b2_, b3_, ones_v):
        c = lax.axis_index("c")
        s = lax.axis_index("s")
        wid = c * NS + s
        g0 = s * G_PER_TILE
        pltpu.sync_copy(zx_hbm.at[pl.ds(g0, G_PER_TILE)], sx.at[pl.ds(g0, G_PER_TILE)])
        pltpu.sync_copy(z64_hbm.at[pl.ds(g0, G_PER_TILE)], s1.at[pl.ds(g0, G_PER_TILE)])
        pltpu.sync_copy(z32_hbm.at[pl.ds(g0, G_PER_TILE)], s2.at[pl.ds(g0, G_PER_TILE)])
        pltpu.sync_copy(z16_hbm.at[pl.ds(g0, G_PER_TILE)], s3.at[pl.ds(g0, G_PER_TILE)])
        pltpu.sync_copy(z16_hbm.at[pl.ds(g0, G_PER_TILE)], scnt.at[pl.ds(g0, G_PER_TILE)])
        pltpu.sync_copy(batch_hbm.at[c, s], batch_v)
        pltpu.sync_copy(ones_hbm, ones_v)
        plsc.subcore_barrier()

        @pl.loop(0, ANB)
        def _(j):
            r = wid * (ANB * AB) + j * AB
            pltpu.sync_copy(x_hbm.at[pl.ds(r, AB)], bx)
            pltpu.sync_copy(bx, sx.at[batch_v.at[j]], add=True)
            pltpu.sync_copy(h1_hbm.at[pl.ds(r, AB)], b1_)
            pltpu.sync_copy(b1_, s1.at[batch_v.at[j]], add=True)
            pltpu.sync_copy(h2_hbm.at[pl.ds(r, AB)], b2_)
            pltpu.sync_copy(b2_, s2.at[batch_v.at[j]], add=True)
            pltpu.sync_copy(h3_hbm.at[pl.ds(r, AB)], b3_)
            pltpu.sync_copy(b3_, s3.at[batch_v.at[j]], add=True)
            pltpu.sync_copy(ones_v, scnt.at[batch_v.at[j]], add=True)

        plsc.subcore_barrier()
        gsl = pl.ds(g0, G_PER_TILE)
        pltpu.sync_copy(sx.at[gsl], ox.at[c, gsl])
        pltpu.sync_copy(s1.at[gsl], o1.at[c, gsl])
        pltpu.sync_copy(s2.at[gsl], o2.at[c, gsl])
        pltpu.sync_copy(s3.at[gsl], o3.at[c, gsl])
        pltpu.sync_copy(scnt.at[gsl], ocnt.at[c, gsl])

    return _agg_kernel


def _mm1_body(x_ref, w_ref, wx_ref, degp_ref, o_ref, oxp_ref):
    x = x_ref[...]
    deg = degp_ref[0, :, 0:1] + degp_ref[1, :, 0:1] + 1.0
    o_ref[...] = lax.rsqrt(deg) * jnp.dot(x, w_ref[...],
                                          preferred_element_type=f32)
    oxp_ref[...] = jnp.dot(x, wx_ref[...], preferred_element_type=f32)


def _mm1(x, w, wx, degp):
    return pl.pallas_call(
        _mm1_body,
        grid=(NBLK,),
        in_specs=[pl.BlockSpec((BLK, F_IN), lambda i: (i, 0)),
                  pl.BlockSpec((F_IN, 64), lambda i: (0, 0)),
                  pl.BlockSpec((F_IN, 64), lambda i: (0, 0)),
                  pl.BlockSpec((2, BLK, 16), lambda i: (0, i, 0))],
        out_specs=[pl.BlockSpec((BLK, 64), lambda i: (i, 0)),
                   pl.BlockSpec((BLK, 64), lambda i: (i, 0))],
        out_shape=[jax.ShapeDtypeStruct((N_PAD, 64), f32),
                   jax.ShapeDtypeStruct((N_PAD, 64), f32)],
    )(x, w, wx, degp)


def _consume_mm_body(acc_ref, y_ref, degp_ref, b_ref, w_ref, h_ref, yn_ref):
    deg = degp_ref[0, :, 0:1] + degp_ref[1, :, 0:1] + 1.0
    dinv = lax.rsqrt(deg)
    h = jnp.maximum(dinv * (acc_ref[0] + acc_ref[1] + y_ref[...]) + b_ref[...],
                    0.0)
    h_ref[...] = h
    yn_ref[...] = dinv * jnp.dot(h, w_ref[...], preferred_element_type=f32)


def _consume_mm(acc, y, degp, b, wt):
    d = y.shape[1]
    dn = wt.shape[1]
    return pl.pallas_call(
        _consume_mm_body,
        grid=(NBLK,),
        in_specs=[pl.BlockSpec((2, BLK, d), lambda i: (0, i, 0)),
                  pl.BlockSpec((BLK, d), lambda i: (i, 0)),
                  pl.BlockSpec((2, BLK, 16), lambda i: (0, i, 0)),
                  pl.BlockSpec((1, d), lambda i: (0, 0)),
                  pl.BlockSpec((d, dn), lambda i: (0, 0))],
        out_specs=[pl.BlockSpec((BLK, d), lambda i: (i, 0)),
                   pl.BlockSpec((BLK, dn), lambda i: (i, 0))],
        out_shape=[jax.ShapeDtypeStruct((N_PAD, d), f32),
                   jax.ShapeDtypeStruct((N_PAD, dn), f32)],
    )(acc, y, degp, b, wt)


def _consume_body(acc_ref, y_ref, degp_ref, b_ref, h_ref):
    deg = degp_ref[0, :, 0:1] + degp_ref[1, :, 0:1] + 1.0
    dinv = lax.rsqrt(deg)
    h_ref[...] = jnp.maximum(
        dinv * (acc_ref[0] + acc_ref[1] + y_ref[...]) + b_ref[...], 0.0)


def _consume(acc, y, degp, b):
    d = y.shape[1]
    return pl.pallas_call(
        _consume_body,
        grid=(NBLK,),
        in_specs=[pl.BlockSpec((2, BLK, d), lambda i: (0, i, 0)),
                  pl.BlockSpec((BLK, d), lambda i: (i, 0)),
                  pl.BlockSpec((2, BLK, 16), lambda i: (0, i, 0)),
                  pl.BlockSpec((1, d), lambda i: (0, 0))],
        out_specs=pl.BlockSpec((BLK, d), lambda i: (i, 0)),
        out_shape=jax.ShapeDtypeStruct((N_PAD, d), f32),
    )(acc, y, degp, b)


def _head_body(axp, a1, a2, a3, acnt, wa, wb, wc, bf1, w2, bf2, o_ref):
    cnt = acnt[0, :, 0:1] + acnt[1, :, 0:1]
    scale = (1.0 / jnp.sqrt(1.0 + 1e-5)) / jnp.maximum(cnt, 1.0)
    m1 = (a1[0] + a1[1]) * scale
    m2 = (a2[0] + a2[1]) * scale
    m3 = (a3[0] + a3[1]) * scale
    z = ((axp[0] + axp[1]) * scale
         + jnp.dot(m1, wa[...], preferred_element_type=f32)
         + jnp.dot(m2, wb[...], preferred_element_type=f32)
         + jnp.dot(m3, wc[...], preferred_element_type=f32)
         + bf1[...])
    z = jnp.maximum(z, 0.0)
    o_ref[...] = jax.nn.sigmoid(
        jnp.dot(z, w2[...], preferred_element_type=f32) + bf2[...])


def _head(axp, a1, a2, a3, acnt, wa, wb, wc, bf1, w2, bf2):
    full = lambda shape: pl.BlockSpec(shape, lambda: tuple(0 for _ in shape))
    args = (axp, a1, a2, a3, acnt, wa, wb, wc, bf1, w2, bf2)
    return pl.pallas_call(
        _head_body,
        in_specs=[full(a.shape) for a in args],
        out_specs=full((G_PAD, 1)),
        out_shape=jax.ShapeDtypeStruct((G_PAD, 1), f32),
    )(*args)


def kernel(x, edge_index, batch, W1, b1, W2, b2, W3, b3, Wfc1, bfc1, Wfc2,
           bfc2):
    w1t = W1.T

    srcs = edge_index[0].reshape(NC, NS, ENB, EB)
    dsts = edge_index[1].reshape(NC, NS, ENB, EB)
    batch_p = (jnp.full((N_PAD,), G, i32).at[:N].set(batch)
               .reshape(NC, NS, ANB, AB))

    ones_e = jnp.ones((EB, 16), f32)
    ones_a = jnp.ones((AB, 16), f32)
    z64 = jnp.zeros((N_PAD, 64), f32)
    z32 = jnp.zeros((N_PAD, 32), f32)
    z16 = jnp.zeros((N_PAD, 16), f32)
    zg64 = jnp.zeros((G_PAD, 64), f32)
    zg32 = jnp.zeros((G_PAD, 32), f32)
    zg16 = jnp.zeros((G_PAD, 16), f32)

    wx = Wfc1[:, :F_IN].T

    degp = _deg_kernel_fn()(dsts, ones_e, z16)
    y1, xp = _mm1(x, w1t, wx, degp)
    acc1 = _make_edge_scatter(64)(y1, srcs, dsts, z64)
    h1, y2 = _consume_mm(acc1, y1, degp, b1.reshape(1, 64), W2.T)
    acc2 = _make_edge_scatter(32)(y2, srcs, dsts, z32)
    h2, y3 = _consume_mm(acc2, y2, degp, b2.reshape(1, 32), W3.T)
    acc3 = _make_edge_scatter(16)(y3, srcs, dsts, z16)
    h3 = _consume(acc3, y3, degp, b3.reshape(1, 16))

    axp, a1, a2, a3, acnt = _agg_kernel_fn()(
        xp, h1, h2, h3, batch_p, ones_a, zg64, zg64, zg32, zg16)

    wa = Wfc1[:, F_IN:F_IN + 64].T
    wb = Wfc1[:, F_IN + 64:F_IN + 96].T
    wc = Wfc1[:, F_IN + 96:].T
    out = _head(axp, a1, a2, a3, acnt, wa, wb, wc,
                bfc1.reshape(1, 64), Wfc2.T, bfc2.reshape(1, 1))
    return out[:G]


# speedup vs baseline: 20.9439x; 1.1207x over previous
import functools

import jax
import jax.numpy as jnp
from jax import lax
from jax.experimental import pallas as pl
from jax.experimental.pallas import tpu as pltpu
from jax.experimental.pallas import tpu_sc as plsc

f32 = jnp.float32
i32 = jnp.int32

N = 9990
E = 140000
G = 90
F_IN = 557

N_PAD = 10240
G_PAD = 96
NC = 2
NS = 16
EB = 125
ENB = 35
E_CORE = NS * ENB * EB
AB = 64
ANB = 5
ROWS_PER_TILE = N_PAD // NS
G_PER_TILE = G_PAD // NS
BLK = 1024
NBLK = N_PAD // BLK


def _sc_mesh():
    return plsc.VectorSubcoreMesh(
        core_axis_name="c", subcore_axis_name="s",
        num_cores=NC, num_subcores=NS)


def _sc_params():
    return pltpu.CompilerParams(use_tc_tiling_on_sc=False)


@functools.cache
def _deg_kernel_fn():
    @functools.partial(
        pl.kernel,
        out_type=jax.ShapeDtypeStruct((NC, N_PAD, 16), f32),
        mesh=_sc_mesh(),
        compiler_params=_sc_params(),
        scratch_types=[
            pltpu.VMEM_SHARED((N_PAD, 16), f32),
            pltpu.VMEM((ENB, EB), i32),
            pltpu.VMEM((EB, 16), f32),
        ],
    )
    def _deg_kernel(dst_hbm, ones_hbm, zeros_hbm, out_hbm, acc, dst_v, ones_v):
        c = lax.axis_index("c")
        s = lax.axis_index("s")
        r0 = s * ROWS_PER_TILE
        pltpu.sync_copy(zeros_hbm.at[pl.ds(r0, ROWS_PER_TILE)],
                        acc.at[pl.ds(r0, ROWS_PER_TILE)])
        pltpu.sync_copy(dst_hbm.at[c, s], dst_v)
        pltpu.sync_copy(ones_hbm, ones_v)
        plsc.subcore_barrier()

        @pl.loop(0, ENB)
        def _(j):
            pltpu.sync_copy(ones_v, acc.at[dst_v.at[j]], add=True)

        plsc.subcore_barrier()
        pltpu.sync_copy(acc.at[pl.ds(r0, ROWS_PER_TILE)],
                        out_hbm.at[c, pl.ds(r0, ROWS_PER_TILE)])

    return _deg_kernel


@functools.cache
def _make_edge_scatter(D):
    @functools.partial(
        pl.kernel,
        out_type=jax.ShapeDtypeStruct((NC, N_PAD, D), f32),
        mesh=_sc_mesh(),
        compiler_params=_sc_params(),
        scratch_types=[
            pltpu.VMEM_SHARED((N_PAD, D), f32),
            pltpu.VMEM_SHARED((N_PAD, D), f32),
            pltpu.VMEM((ENB, EB), i32),
            pltpu.VMEM((ENB, EB), i32),
            pltpu.VMEM((EB, D), f32),
        ],
    )
    def k(y_hbm, src_hbm, dst_hbm, zeros_hbm, out_hbm, acc, y_sh, src_v,
          dst_v, rows_v):
        c = lax.axis_index("c")
        s = lax.axis_index("s")
        r0 = s * ROWS_PER_TILE
        pltpu.sync_copy(zeros_hbm.at[pl.ds(r0, ROWS_PER_TILE)],
                        acc.at[pl.ds(r0, ROWS_PER_TILE)])
        pltpu.sync_copy(y_hbm.at[pl.ds(r0, ROWS_PER_TILE)],
                        y_sh.at[pl.ds(r0, ROWS_PER_TILE)])
        pltpu.sync_copy(src_hbm.at[c, s], src_v)
        pltpu.sync_copy(dst_hbm.at[c, s], dst_v)
        plsc.subcore_barrier()

        @pl.loop(0, ENB)
        def _(j):
            pltpu.sync_copy(y_sh.at[src_v.at[j]], rows_v)
            pltpu.sync_copy(rows_v, acc.at[dst_v.at[j]], add=True)

        plsc.subcore_barrier()
        pltpu.sync_copy(acc.at[pl.ds(r0, ROWS_PER_TILE)],
                        out_hbm.at[c, pl.ds(r0, ROWS_PER_TILE)])

    return k


@functools.cache
def _agg_kernel_fn():
    @functools.partial(
        pl.kernel,
        out_type=[
            jax.ShapeDtypeStruct((NC, G_PAD, 64), f32),
            jax.ShapeDtypeStruct((NC, G_PAD, 64), f32),
            jax.ShapeDtypeStruct((NC, G_PAD, 32), f32),
            jax.ShapeDtypeStruct((NC, G_PAD, 16), f32),
            jax.ShapeDtypeStruct((NC, G_PAD, 16), f32),
        ],
        mesh=_sc_mesh(),
        compiler_params=_sc_params(),
        scratch_types=[
        pltpu.VMEM_SHARED((G_PAD, 64), f32),
        pltpu.VMEM_SHARED((G_PAD, 64), f32),
        pltpu.VMEM_SHARED((G_PAD, 32), f32),
        pltpu.VMEM_SHARED((G_PAD, 16), f32),
        pltpu.VMEM_SHARED((G_PAD, 16), f32),
        pltpu.VMEM((ANB, AB), i32),
        pltpu.VMEM((AB, 64), f32),
        pltpu.VMEM((AB, 64), f32),
        pltpu.VMEM((AB, 32), f32),
        pltpu.VMEM((AB, 16), f32),
        pltpu.VMEM((AB, 16), f32),
    ],
)
    def _agg_kernel(x_hbm, h1_hbm, h2_hbm, h3_hbm, batch_hbm, ones_hbm,
                    zx_hbm, z64_hbm, z32_hbm, z16_hbm,
                    ox, o1, o2, o3, ocnt,
                    sx, s1, s2, s3, scnt, batch_v, bx, b1_, b2_, b3_, ones_v):
        c = lax.axis_index("c")
        s = lax.axis_index("s")
        wid = c * NS + s
        g0 = s * G_PER_TILE
        pltpu.sync_copy(zx_hbm.at[pl.ds(g0, G_PER_TILE)], sx.at[pl.ds(g0, G_PER_TILE)])
        pltpu.sync_copy(z64_hbm.at[pl.ds(g0, G_PER_TILE)], s1.at[pl.ds(g0, G_PER_TILE)])
        pltpu.sync_copy(z32_hbm.at[pl.ds(g0, G_PER_TILE)], s2.at[pl.ds(g0, G_PER_TILE)])
        pltpu.sync_copy(z16_hbm.at[pl.ds(g0, G_PER_TILE)], s3.at[pl.ds(g0, G_PER_TILE)])
        pltpu.sync_copy(z16_hbm.at[pl.ds(g0, G_PER_TILE)], scnt.at[pl.ds(g0, G_PER_TILE)])
        pltpu.sync_copy(batch_hbm.at[c, s], batch_v)
        pltpu.sync_copy(ones_hbm, ones_v)
        plsc.subcore_barrier()

        @pl.loop(0, ANB)
        def _(j):
            r = wid * (ANB * AB) + j * AB
            pltpu.sync_copy(x_hbm.at[pl.ds(r, AB)], bx)
            pltpu.sync_copy(bx, sx.at[batch_v.at[j]], add=True)
            pltpu.sync_copy(h1_hbm.at[pl.ds(r, AB)], b1_)
            pltpu.sync_copy(b1_, s1.at[batch_v.at[j]], add=True)
            pltpu.sync_copy(h2_hbm.at[pl.ds(r, AB)], b2_)
            pltpu.sync_copy(b2_, s2.at[batch_v.at[j]], add=True)
            pltpu.sync_copy(h3_hbm.at[pl.ds(r, AB)], b3_)
            pltpu.sync_copy(b3_, s3.at[batch_v.at[j]], add=True)
            pltpu.sync_copy(ones_v, scnt.at[batch_v.at[j]], add=True)

        plsc.subcore_barrier()
        gsl = pl.ds(g0, G_PER_TILE)
        pltpu.sync_copy(sx.at[gsl], ox.at[c, gsl])
        pltpu.sync_copy(s1.at[gsl], o1.at[c, gsl])
        pltpu.sync_copy(s2.at[gsl], o2.at[c, gsl])
        pltpu.sync_copy(s3.at[gsl], o3.at[c, gsl])
        pltpu.sync_copy(scnt.at[gsl], ocnt.at[c, gsl])

    return _agg_kernel


def _mm1_body(x_ref, w_ref, wx_ref, degp_ref, o_ref, oxp_ref):
    x = x_ref[...]
    deg = degp_ref[0, :, 0:1] + degp_ref[1, :, 0:1] + 1.0
    o_ref[...] = lax.rsqrt(deg) * jnp.dot(x, w_ref[...],
                                          preferred_element_type=f32)
    oxp_ref[...] = jnp.dot(x, wx_ref[...], preferred_element_type=f32)


def _mm1(x, w, wx, degp):
    return pl.pallas_call(
        _mm1_body,
        grid=(NBLK,),
        in_specs=[pl.BlockSpec((BLK, F_IN), lambda i: (i, 0)),
                  pl.BlockSpec((F_IN, 64), lambda i: (0, 0)),
                  pl.BlockSpec((F_IN, 64), lambda i: (0, 0)),
                  pl.BlockSpec((2, BLK, 16), lambda i: (0, i, 0))],
        out_specs=[pl.BlockSpec((BLK, 64), lambda i: (i, 0)),
                   pl.BlockSpec((BLK, 64), lambda i: (i, 0))],
        out_shape=[jax.ShapeDtypeStruct((N_PAD, 64), f32),
                   jax.ShapeDtypeStruct((N_PAD, 64), f32)],
    )(x, w, wx, degp)


def _consume_mm_body(acc_ref, y_ref, degp_ref, b_ref, w_ref, h_ref, yn_ref):
    deg = degp_ref[0, :, 0:1] + degp_ref[1, :, 0:1] + 1.0
    dinv = lax.rsqrt(deg)
    h = jnp.maximum(dinv * (acc_ref[0] + acc_ref[1] + y_ref[...]) + b_ref[...],
                    0.0)
    h_ref[...] = h
    yn_ref[...] = dinv * jnp.dot(h, w_ref[...], preferred_element_type=f32)


def _consume_mm(acc, y, degp, b, wt):
    d = y.shape[1]
    dn = wt.shape[1]
    return pl.pallas_call(
        _consume_mm_body,
        grid=(NBLK,),
        in_specs=[pl.BlockSpec((2, BLK, d), lambda i: (0, i, 0)),
                  pl.BlockSpec((BLK, d), lambda i: (i, 0)),
                  pl.BlockSpec((2, BLK, 16), lambda i: (0, i, 0)),
                  pl.BlockSpec((1, d), lambda i: (0, 0)),
                  pl.BlockSpec((d, dn), lambda i: (0, 0))],
        out_specs=[pl.BlockSpec((BLK, d), lambda i: (i, 0)),
                   pl.BlockSpec((BLK, dn), lambda i: (i, 0))],
        out_shape=[jax.ShapeDtypeStruct((N_PAD, d), f32),
                   jax.ShapeDtypeStruct((N_PAD, dn), f32)],
    )(acc, y, degp, b, wt)


def _consume_body(acc_ref, y_ref, degp_ref, b_ref, h_ref):
    deg = degp_ref[0, :, 0:1] + degp_ref[1, :, 0:1] + 1.0
    dinv = lax.rsqrt(deg)
    h_ref[...] = jnp.maximum(
        dinv * (acc_ref[0] + acc_ref[1] + y_ref[...]) + b_ref[...], 0.0)


def _consume(acc, y, degp, b):
    d = y.shape[1]
    return pl.pallas_call(
        _consume_body,
        grid=(NBLK,),
        in_specs=[pl.BlockSpec((2, BLK, d), lambda i: (0, i, 0)),
                  pl.BlockSpec((BLK, d), lambda i: (i, 0)),
                  pl.BlockSpec((2, BLK, 16), lambda i: (0, i, 0)),
                  pl.BlockSpec((1, d), lambda i: (0, 0))],
        out_specs=pl.BlockSpec((BLK, d), lambda i: (i, 0)),
        out_shape=jax.ShapeDtypeStruct((N_PAD, d), f32),
    )(acc, y, degp, b)


def _head_body(axp, a1, a2, a3, acnt, wa, wb, wc, bf1, w2, bf2, o_ref):
    cnt = acnt[0, :, 0:1] + acnt[1, :, 0:1]
    scale = (1.0 / jnp.sqrt(1.0 + 1e-5)) / jnp.maximum(cnt, 1.0)
    m1 = (a1[0] + a1[1]) * scale
    m2 = (a2[0] + a2[1]) * scale
    m3 = (a3[0] + a3[1]) * scale
    z = ((axp[0] + axp[1]) * scale
         + jnp.dot(m1, wa[...], preferred_element_type=f32)
         + jnp.dot(m2, wb[...], preferred_element_type=f32)
         + jnp.dot(m3, wc[...], preferred_element_type=f32)
         + bf1[...])
    z = jnp.maximum(z, 0.0)
    o_ref[...] = jax.nn.sigmoid(
        jnp.dot(z, w2[...], preferred_element_type=f32) + bf2[...])


def _head(axp, a1, a2, a3, acnt, wa, wb, wc, bf1, w2, bf2):
    full = lambda shape: pl.BlockSpec(shape, lambda: tuple(0 for _ in shape))
    args = (axp, a1, a2, a3, acnt, wa, wb, wc, bf1, w2, bf2)
    return pl.pallas_call(
        _head_body,
        in_specs=[full(a.shape) for a in args],
        out_specs=full((G_PAD, 1)),
        out_shape=jax.ShapeDtypeStruct((G_PAD, 1), f32),
    )(*args)


def kernel(x, edge_index, batch, W1, b1, W2, b2, W3, b3, Wfc1, bfc1, Wfc2,
           bfc2):
    w1t = W1.T

    srcs = edge_index[0].reshape(NC, NS, ENB, EB)
    dsts = edge_index[1].reshape(NC, NS, ENB, EB)
    batch_p = (jnp.full((N_PAD,), G, i32).at[:N].set(batch)
               .reshape(NC, NS, ANB, AB))

    ones_e = jnp.ones((EB, 16), f32)
    ones_a = jnp.ones((AB, 16), f32)
    z64 = jnp.zeros((N_PAD, 64), f32)
    z32 = jnp.zeros((N_PAD, 32), f32)
    z16 = jnp.zeros((N_PAD, 16), f32)
    zg64 = jnp.zeros((G_PAD, 64), f32)
    zg32 = jnp.zeros((G_PAD, 32), f32)
    zg16 = jnp.zeros((G_PAD, 16), f32)

    wx = Wfc1[:, :F_IN].T

    degp = _deg_kernel_fn()(dsts, ones_e, z16)
    y1, xp = _mm1(x, w1t, wx, degp)
    acc1 = _make_edge_scatter(64)(y1, srcs, dsts, z64)
    h1, y2 = _consume_mm(acc1, y1, degp, b1.reshape(1, 64), W2.T)
    acc2 = _make_edge_scatter(32)(y2, srcs, dsts, z32)
    h2, y3 = _consume_mm(acc2, y2, degp, b2.reshape(1, 32), W3.T)
    acc3 = _make_edge_scatter(16)(y3, srcs, dsts, z16)
    h3 = _consume(acc3, y3, degp, b3.reshape(1, 16))

    axp, a1, a2, a3, acnt = _agg_kernel_fn()(
        xp, h1, h2, h3, batch_p, ones_a, zg64, zg64, zg32, zg16)

    wa = Wfc1[:, F_IN:F_IN + 64].T
    wb = Wfc1[:, F_IN + 64:F_IN + 96].T
    wc = Wfc1[:, F_IN + 96:].T
    out = _head(axp, a1, a2, a3, acnt, wa, wb, wc,
                bfc1.reshape(1, 64), Wfc2.T, bfc2.reshape(1, 1))
    return out[:G]


# speedup vs baseline: 21.3114x; 1.0175x over previous
import functools

import jax
import jax.numpy as jnp
from jax import lax
from jax.experimental import pallas as pl
from jax.experimental.pallas import tpu as pltpu
from jax.experimental.pallas import tpu_sc as plsc

f32 = jnp.float32
i32 = jnp.int32

N = 9990
E = 140000
G = 90
F_IN = 557

N_PAD = 10240
G_PAD = 96
NC = 2
NS = 16
EB = 125
ENB = 35
E_CORE = NS * ENB * EB
AB = 64
ANB = 5
ROWS_PER_TILE = N_PAD // NS
G_PER_TILE = G_PAD // NS
BLK = 1024
NBLK = N_PAD // BLK
WF = 176


def _sc_mesh():
    return plsc.VectorSubcoreMesh(
        core_axis_name="c", subcore_axis_name="s",
        num_cores=NC, num_subcores=NS)


def _sc_params():
    return pltpu.CompilerParams(use_tc_tiling_on_sc=False)


@functools.cache
def _deg_kernel_fn():
    @functools.partial(
        pl.kernel,
        out_type=jax.ShapeDtypeStruct((NC, N_PAD, 16), f32),
        mesh=_sc_mesh(),
        compiler_params=_sc_params(),
        scratch_types=[
            pltpu.VMEM_SHARED((N_PAD, 16), f32),
            pltpu.VMEM((ENB, EB), i32),
            pltpu.VMEM((EB, 16), f32),
        ],
    )
    def _deg_kernel(dst_hbm, ones_hbm, zeros_hbm, out_hbm, acc, dst_v, ones_v):
        c = lax.axis_index("c")
        s = lax.axis_index("s")
        r0 = s * ROWS_PER_TILE
        pltpu.sync_copy(zeros_hbm.at[pl.ds(r0, ROWS_PER_TILE)],
                        acc.at[pl.ds(r0, ROWS_PER_TILE)])
        pltpu.sync_copy(dst_hbm.at[c, s], dst_v)
        pltpu.sync_copy(ones_hbm, ones_v)
        plsc.subcore_barrier()

        @pl.loop(0, ENB)
        def _(j):
            pltpu.sync_copy(ones_v, acc.at[dst_v.at[j]], add=True)

        plsc.subcore_barrier()
        pltpu.sync_copy(acc.at[pl.ds(r0, ROWS_PER_TILE)],
                        out_hbm.at[c, pl.ds(r0, ROWS_PER_TILE)])

    return _deg_kernel


@functools.cache
def _make_edge_scatter(D):
    @functools.partial(
        pl.kernel,
        out_type=jax.ShapeDtypeStruct((NC, N_PAD, D), f32),
        mesh=_sc_mesh(),
        compiler_params=_sc_params(),
        scratch_types=[
            pltpu.VMEM_SHARED((N_PAD, D), f32),
            pltpu.VMEM_SHARED((N_PAD, D), f32),
            pltpu.VMEM((ENB, EB), i32),
            pltpu.VMEM((ENB, EB), i32),
            pltpu.VMEM((EB, D), f32),
        ],
    )
    def k(y_hbm, src_hbm, dst_hbm, zeros_hbm, out_hbm, acc, y_sh, src_v,
          dst_v, rows_v):
        c = lax.axis_index("c")
        s = lax.axis_index("s")
        r0 = s * ROWS_PER_TILE
        pltpu.sync_copy(zeros_hbm.at[pl.ds(r0, ROWS_PER_TILE)],
                        acc.at[pl.ds(r0, ROWS_PER_TILE)])
        pltpu.sync_copy(y_hbm.at[pl.ds(r0, ROWS_PER_TILE)],
                        y_sh.at[pl.ds(r0, ROWS_PER_TILE)])
        pltpu.sync_copy(src_hbm.at[c, s], src_v)
        pltpu.sync_copy(dst_hbm.at[c, s], dst_v)
        plsc.subcore_barrier()

        @pl.loop(0, ENB)
        def _(j):
            pltpu.sync_copy(y_sh.at[src_v.at[j]], rows_v)
            pltpu.sync_copy(rows_v, acc.at[dst_v.at[j]], add=True)

        plsc.subcore_barrier()
        pltpu.sync_copy(acc.at[pl.ds(r0, ROWS_PER_TILE)],
                        out_hbm.at[c, pl.ds(r0, ROWS_PER_TILE)])

    return k


@functools.cache
def _agg_kernel_fn():
    @functools.partial(
        pl.kernel,
        out_type=[
            jax.ShapeDtypeStruct((NC, G_PAD, WF), f32),
            jax.ShapeDtypeStruct((NC, G_PAD, 16), f32),
        ],
        mesh=_sc_mesh(),
        compiler_params=_sc_params(),
        scratch_types=[
            pltpu.VMEM_SHARED((G_PAD, WF), f32),
            pltpu.VMEM_SHARED((G_PAD, 16), f32),
            pltpu.VMEM((ANB, AB), i32),
            pltpu.VMEM((AB, WF), f32),
            pltpu.VMEM((AB, 16), f32),
        ],
    )
    def _agg_kernel(big_hbm, batch_hbm, ones_hbm, zw_hbm, z16_hbm,
                    obig, ocnt, sbig, scnt, batch_v, bbuf, ones_v):
        c = lax.axis_index("c")
        s = lax.axis_index("s")
        wid = c * NS + s
        g0 = s * G_PER_TILE
        pltpu.sync_copy(zw_hbm.at[pl.ds(g0, G_PER_TILE)],
                        sbig.at[pl.ds(g0, G_PER_TILE)])
        pltpu.sync_copy(z16_hbm.at[pl.ds(g0, G_PER_TILE)],
                        scnt.at[pl.ds(g0, G_PER_TILE)])
        pltpu.sync_copy(batch_hbm.at[c, s], batch_v)
        pltpu.sync_copy(ones_hbm, ones_v)
        plsc.subcore_barrier()

        @pl.loop(0, ANB)
        def _(j):
            r = wid * (ANB * AB) + j * AB
            pltpu.sync_copy(big_hbm.at[pl.ds(r, AB)], bbuf)
            pltpu.sync_copy(bbuf, sbig.at[batch_v.at[j]], add=True)
            pltpu.sync_copy(ones_v, scnt.at[batch_v.at[j]], add=True)

        plsc.subcore_barrier()
        gsl = pl.ds(g0, G_PER_TILE)
        pltpu.sync_copy(sbig.at[gsl], obig.at[c, gsl])
        pltpu.sync_copy(scnt.at[gsl], ocnt.at[c, gsl])

    return _agg_kernel


def _mm1_body(x_ref, w_ref, wx_ref, degp_ref, o_ref, oxp_ref):
    x = x_ref[...]
    deg = degp_ref[0, :, 0:1] + degp_ref[1, :, 0:1] + 1.0
    o_ref[...] = lax.rsqrt(deg) * jnp.dot(x, w_ref[...],
                                          preferred_element_type=f32)
    oxp_ref[...] = jnp.dot(x, wx_ref[...], preferred_element_type=f32)


def _mm1(x, w, wx, degp):
    return pl.pallas_call(
        _mm1_body,
        grid=(NBLK,),
        in_specs=[pl.BlockSpec((BLK, F_IN), lambda i: (i, 0)),
                  pl.BlockSpec((F_IN, 64), lambda i: (0, 0)),
                  pl.BlockSpec((F_IN, 64), lambda i: (0, 0)),
                  pl.BlockSpec((2, BLK, 16), lambda i: (0, i, 0))],
        out_specs=[pl.BlockSpec((BLK, 64), lambda i: (i, 0)),
                   pl.BlockSpec((BLK, 64), lambda i: (i, 0))],
        out_shape=[jax.ShapeDtypeStruct((N_PAD, 64), f32),
                   jax.ShapeDtypeStruct((N_PAD, 64), f32)],
    )(x, w, wx, degp)


def _consume_mm_body(acc_ref, y_ref, degp_ref, b_ref, w_ref, h_ref, yn_ref):
    deg = degp_ref[0, :, 0:1] + degp_ref[1, :, 0:1] + 1.0
    dinv = lax.rsqrt(deg)
    h = jnp.maximum(dinv * (acc_ref[0] + acc_ref[1] + y_ref[...]) + b_ref[...],
                    0.0)
    h_ref[...] = h
    yn_ref[...] = dinv * jnp.dot(h, w_ref[...], preferred_element_type=f32)


def _consume_mm(acc, y, degp, b, wt):
    d = y.shape[1]
    dn = wt.shape[1]
    return pl.pallas_call(
        _consume_mm_body,
        grid=(NBLK,),
        in_specs=[pl.BlockSpec((2, BLK, d), lambda i: (0, i, 0)),
                  pl.BlockSpec((BLK, d), lambda i: (i, 0)),
                  pl.BlockSpec((2, BLK, 16), lambda i: (0, i, 0)),
                  pl.BlockSpec((1, d), lambda i: (0, 0)),
                  pl.BlockSpec((d, dn), lambda i: (0, 0))],
        out_specs=[pl.BlockSpec((BLK, d), lambda i: (i, 0)),
                   pl.BlockSpec((BLK, dn), lambda i: (i, 0))],
        out_shape=[jax.ShapeDtypeStruct((N_PAD, d), f32),
                   jax.ShapeDtypeStruct((N_PAD, dn), f32)],
    )(acc, y, degp, b, wt)


def _consume_body(acc_ref, y_ref, degp_ref, b_ref, xp_ref, h1_ref, h2_ref,
                  big_ref):
    deg = degp_ref[0, :, 0:1] + degp_ref[1, :, 0:1] + 1.0
    dinv = lax.rsqrt(deg)
    h3 = jnp.maximum(
        dinv * (acc_ref[0] + acc_ref[1] + y_ref[...]) + b_ref[...], 0.0)
    big_ref[...] = jnp.concatenate(
        [xp_ref[...], h1_ref[...], h2_ref[...], h3], axis=1)


def _consume(acc, y, degp, b, xp, h1, h2):
    d = y.shape[1]
    return pl.pallas_call(
        _consume_body,
        grid=(NBLK,),
        in_specs=[pl.BlockSpec((2, BLK, d), lambda i: (0, i, 0)),
                  pl.BlockSpec((BLK, d), lambda i: (i, 0)),
                  pl.BlockSpec((2, BLK, 16), lambda i: (0, i, 0)),
                  pl.BlockSpec((1, d), lambda i: (0, 0)),
                  pl.BlockSpec((BLK, 64), lambda i: (i, 0)),
                  pl.BlockSpec((BLK, 64), lambda i: (i, 0)),
                  pl.BlockSpec((BLK, 32), lambda i: (i, 0))],
        out_specs=pl.BlockSpec((BLK, WF), lambda i: (i, 0)),
        out_shape=jax.ShapeDtypeStruct((N_PAD, WF), f32),
    )(acc, y, degp, b, xp, h1, h2)


def _head_body(abig, acnt, wa, wb, wc, bf1, w2, bf2, o_ref):
    cnt = acnt[0, :, 0:1] + acnt[1, :, 0:1]
    scale = (1.0 / jnp.sqrt(1.0 + 1e-5)) / jnp.maximum(cnt, 1.0)
    big = (abig[0] + abig[1]) * scale
    z = (big[:, 0:64]
         + jnp.dot(big[:, 64:128], wa[...], preferred_element_type=f32)
         + jnp.dot(big[:, 128:160], wb[...], preferred_element_type=f32)
         + jnp.dot(big[:, 160:176], wc[...], preferred_element_type=f32)
         + bf1[...])
    z = jnp.maximum(z, 0.0)
    o_ref[...] = jax.nn.sigmoid(
        jnp.dot(z, w2[...], preferred_element_type=f32) + bf2[...])


def _head(abig, acnt, wa, wb, wc, bf1, w2, bf2):
    full = lambda shape: pl.BlockSpec(shape, lambda: tuple(0 for _ in shape))
    args = (abig, acnt, wa, wb, wc, bf1, w2, bf2)
    return pl.pallas_call(
        _head_body,
        in_specs=[full(a.shape) for a in args],
        out_specs=full((G_PAD, 1)),
        out_shape=jax.ShapeDtypeStruct((G_PAD, 1), f32),
    )(*args)


def kernel(x, edge_index, batch, W1, b1, W2, b2, W3, b3, Wfc1, bfc1, Wfc2,
           bfc2):
    w1t = W1.T

    srcs = edge_index[0].reshape(NC, NS, ENB, EB)
    dsts = edge_index[1].reshape(NC, NS, ENB, EB)
    batch_p = (jnp.full((N_PAD,), G, i32).at[:N].set(batch)
               .reshape(NC, NS, ANB, AB))

    ones_e = jnp.ones((EB, 16), f32)
    ones_a = jnp.ones((AB, 16), f32)
    z64 = jnp.zeros((N_PAD, 64), f32)
    z32 = jnp.zeros((N_PAD, 32), f32)
    z16 = jnp.zeros((N_PAD, 16), f32)
    zgw = jnp.zeros((G_PAD, WF), f32)
    zg16 = jnp.zeros((G_PAD, 16), f32)

    wx = Wfc1[:, :F_IN].T

    degp = _deg_kernel_fn()(dsts, ones_e, z16)
    y1, xp = _mm1(x, w1t, wx, degp)
    acc1 = _make_edge_scatter(64)(y1, srcs, dsts, z64)
    h1, y2 = _consume_mm(acc1, y1, degp, b1.reshape(1, 64), W2.T)
    acc2 = _make_edge_scatter(32)(y2, srcs, dsts, z32)
    h2, y3 = _consume_mm(acc2, y2, degp, b2.reshape(1, 32), W3.T)
    acc3 = _make_edge_scatter(16)(y3, srcs, dsts, z16)
    big = _consume(acc3, y3, degp, b3.reshape(1, 16), xp, h1, h2)

    abig, acnt = _agg_kernel_fn()(big, batch_p, ones_a, zgw, zg16)

    wa = Wfc1[:, F_IN:F_IN + 64].T
    wb = Wfc1[:, F_IN + 64:F_IN + 96].T
    wc = Wfc1[:, F_IN + 96:].T
    out = _head(abig, acnt, wa, wb, wc,
                bfc1.reshape(1, 64), Wfc2.T, bfc2.reshape(1, 1))
    return out[:G]
